# baseline (speedup 1.0000x reference)
import functools

import jax
import jax.numpy as jnp
import numpy as np
from jax import lax
from jax.experimental import pallas as pl
from jax.experimental.pallas import tpu as pltpu
from jax.experimental.pallas import tpu_sc as plsc

B = 16384
D = 32
NUM_LOC = 1000
NUM_LVL = 10
NC = 2
NS = 16
NW = NC * NS
BPW = B // NW
CHUNK = 128
NCHUNK = BPW // CHUNK

_mesh = plsc.VectorSubcoreMesh(core_axis_name="c", subcore_axis_name="s")

_OUT_IDX = tuple(
    (3 * np.arange(B, dtype=np.int32) + k).reshape(NW * NCHUNK, CHUNK)
    for k in range(3)
)


@functools.partial(
    pl.kernel,
    mesh=_mesh,
    compiler_params=pltpu.CompilerParams(use_tc_tiling_on_sc=False),
    out_type=jax.ShapeDtypeStruct((3 * B, D), jnp.float32),
    scratch_types=[
        pltpu.VMEM((NCHUNK, CHUNK), jnp.int32),
        pltpu.VMEM((NCHUNK, CHUNK), jnp.int32),
        pltpu.VMEM((NCHUNK, CHUNK), jnp.int32),
        pltpu.VMEM((NCHUNK, CHUNK), jnp.int32),
        pltpu.VMEM((NCHUNK, CHUNK), jnp.int32),
        pltpu.VMEM((NCHUNK, CHUNK), jnp.int32),
        pltpu.VMEM((BPW, D), jnp.float32),
        pltpu.VMEM((BPW, D), jnp.float32),
        pltpu.VMEM((BPW, D), jnp.float32),
        pltpu.VMEM_SHARED((NUM_LOC + 1, D), jnp.float32),
        pltpu.VMEM_SHARED((NUM_LVL + 1, D), jnp.float32),
        pltpu.SemaphoreType.DMA,
        pltpu.SemaphoreType.DMA,
    ],
)
def _emb_kernel(title_idx_hbm, loc_idx_hbm, lvl_idx_hbm,
                oidx_t_hbm, oidx_l_hbm, oidx_v_hbm,
                title_tab, loc_tab, lvl_tab, out_hbm,
                tidx_v, lidx_v, vidx_v, toidx_v, loidx_v, voidx_v,
                trows, lrows, vrows, loc_sp, lvl_sp, sem_g, sem_s):
    wid = lax.axis_index("s") * NC + lax.axis_index("c")
    row0 = wid * NCHUNK

    pltpu.sync_copy(title_idx_hbm.at[pl.ds(row0, NCHUNK)], tidx_v)
    pltpu.sync_copy(loc_idx_hbm.at[pl.ds(row0, NCHUNK)], lidx_v)
    pltpu.sync_copy(lvl_idx_hbm.at[pl.ds(row0, NCHUNK)], vidx_v)
    pltpu.sync_copy(oidx_t_hbm.at[pl.ds(row0, NCHUNK)], toidx_v)
    pltpu.sync_copy(oidx_l_hbm.at[pl.ds(row0, NCHUNK)], loidx_v)
    pltpu.sync_copy(oidx_v_hbm.at[pl.ds(row0, NCHUNK)], voidx_v)

    gathers = []
    for j in range(NCHUNK):
        gathers.append(pltpu.async_copy(
            title_tab.at[tidx_v.at[j]], trows.at[pl.ds(j * CHUNK, CHUNK)], sem_g))

    @pl.when(lax.axis_index("s") == 0)
    def _stage():
        pltpu.sync_copy(loc_tab, loc_sp)
        pltpu.sync_copy(lvl_tab, lvl_sp)
    plsc.subcore_barrier()

    for j in range(NCHUNK):
        gathers.append(pltpu.async_copy(
            loc_sp.at[lidx_v.at[j]], lrows.at[pl.ds(j * CHUNK, CHUNK)], sem_g))
        gathers.append(pltpu.async_copy(
            lvl_sp.at[vidx_v.at[j]], vrows.at[pl.ds(j * CHUNK, CHUNK)], sem_g))
    for c in gathers:
        c.wait()

    scatters = []
    for j in range(NCHUNK):
        scatters.append(pltpu.async_copy(
            trows.at[pl.ds(j * CHUNK, CHUNK)], out_hbm.at[toidx_v.at[j]], sem_s))
        scatters.append(pltpu.async_copy(
            lrows.at[pl.ds(j * CHUNK, CHUNK)], out_hbm.at[loidx_v.at[j]], sem_s))
        scatters.append(pltpu.async_copy(
            vrows.at[pl.ds(j * CHUNK, CHUNK)], out_hbm.at[voidx_v.at[j]], sem_s))
    for c in scatters:
        c.wait()


def kernel(movie_title, location, level, title_table, location_table, level_table):
    t_idx = movie_title.astype(jnp.int32).reshape(NW * NCHUNK, CHUNK)
    l_idx = location.astype(jnp.int32).reshape(NW * NCHUNK, CHUNK)
    v_idx = level.astype(jnp.int32).reshape(NW * NCHUNK, CHUNK)
    out = _emb_kernel(t_idx, l_idx, v_idx,
                      _OUT_IDX[0], _OUT_IDX[1], _OUT_IDX[2],
                      title_table, location_table, level_table)
    return out


# speedup vs baseline: 1.1056x; 1.1056x over previous
import functools

import jax
import jax.numpy as jnp
import numpy as np
from jax import lax
from jax.experimental import pallas as pl
from jax.experimental.pallas import tpu as pltpu
from jax.experimental.pallas import tpu_sc as plsc

B = 16384
D = 32
NUM_LOC = 1000
NUM_LVL = 10
NC = 2
NS = 16
NW = NC * NS
BPW = B // NW
CHUNK = 128
NCHUNK = BPW // CHUNK

_mesh = plsc.VectorSubcoreMesh(core_axis_name="c", subcore_axis_name="s")

_OUT_IDX = tuple(
    (3 * np.arange(B, dtype=np.int32) + k).reshape(NW * NCHUNK, CHUNK)
    for k in range(3)
)


@functools.partial(
    pl.kernel,
    mesh=_mesh,
    compiler_params=pltpu.CompilerParams(use_tc_tiling_on_sc=False),
    out_type=jax.ShapeDtypeStruct((3 * B, D), jnp.float32),
    scratch_types=[
        pltpu.VMEM((BPW,), jnp.int32),
        pltpu.VMEM((BPW,), jnp.int32),
        pltpu.VMEM((BPW,), jnp.int32),
        pltpu.VMEM((NCHUNK, CHUNK), jnp.int32),
        pltpu.VMEM((NCHUNK, CHUNK), jnp.int32),
        pltpu.VMEM((NCHUNK, CHUNK), jnp.int32),
        pltpu.VMEM((BPW, D), jnp.float32),
        pltpu.VMEM((BPW, D), jnp.float32),
        pltpu.VMEM((BPW, D), jnp.float32),
        pltpu.VMEM_SHARED((NUM_LOC + 1, D), jnp.float32),
        pltpu.VMEM_SHARED((NUM_LVL + 1, D), jnp.float32),
        pltpu.SemaphoreType.DMA,
        pltpu.SemaphoreType.DMA,
    ],
)
def _emb_kernel(title_idx_hbm, loc_idx_hbm, lvl_idx_hbm,
                oidx_t_hbm, oidx_l_hbm, oidx_v_hbm,
                title_tab, loc_tab, lvl_tab, out_hbm,
                tidx_v, lidx_v, vidx_v, toidx_v, loidx_v, voidx_v,
                trows, lrows, vrows, loc_sp, lvl_sp, sem_g, sem_s):
    wid = lax.axis_index("s") * NC + lax.axis_index("c")
    base = wid * BPW
    row0 = wid * NCHUNK

    pltpu.sync_copy(title_idx_hbm.at[pl.ds(base, BPW)], tidx_v)
    pltpu.sync_copy(loc_idx_hbm.at[pl.ds(base, BPW)], lidx_v)
    pltpu.sync_copy(lvl_idx_hbm.at[pl.ds(base, BPW)], vidx_v)
    pltpu.sync_copy(oidx_t_hbm.at[pl.ds(row0, NCHUNK)], toidx_v)
    pltpu.sync_copy(oidx_l_hbm.at[pl.ds(row0, NCHUNK)], loidx_v)
    pltpu.sync_copy(oidx_v_hbm.at[pl.ds(row0, NCHUNK)], voidx_v)

    gathers = []
    for j in range(NCHUNK):
        gathers.append(pltpu.async_copy(
            title_tab.at[tidx_v.at[pl.ds(j * CHUNK, CHUNK)]],
            trows.at[pl.ds(j * CHUNK, CHUNK)], sem_g))

    @pl.when(lax.axis_index("s") == 0)
    def _stage():
        pltpu.sync_copy(loc_tab, loc_sp)
        pltpu.sync_copy(lvl_tab, lvl_sp)
    plsc.subcore_barrier()

    for j in range(NCHUNK):
        gathers.append(pltpu.async_copy(
            loc_sp.at[lidx_v.at[pl.ds(j * CHUNK, CHUNK)]],
            lrows.at[pl.ds(j * CHUNK, CHUNK)], sem_g))
        gathers.append(pltpu.async_copy(
            lvl_sp.at[vidx_v.at[pl.ds(j * CHUNK, CHUNK)]],
            vrows.at[pl.ds(j * CHUNK, CHUNK)], sem_g))
    for c in gathers:
        c.wait()

    scatters = []
    for j in range(NCHUNK):
        scatters.append(pltpu.async_copy(
            trows.at[pl.ds(j * CHUNK, CHUNK)], out_hbm.at[toidx_v.at[j]], sem_s))
        scatters.append(pltpu.async_copy(
            lrows.at[pl.ds(j * CHUNK, CHUNK)], out_hbm.at[loidx_v.at[j]], sem_s))
        scatters.append(pltpu.async_copy(
            vrows.at[pl.ds(j * CHUNK, CHUNK)], out_hbm.at[voidx_v.at[j]], sem_s))
    for c in scatters:
        c.wait()


def kernel(movie_title, location, level, title_table, location_table, level_table):
    t_idx = movie_title.astype(jnp.int32)
    l_idx = location.astype(jnp.int32)
    v_idx = level.astype(jnp.int32)
    out = _emb_kernel(t_idx, l_idx, v_idx,
                      _OUT_IDX[0], _OUT_IDX[1], _OUT_IDX[2],
                      title_table, location_table, level_table)
    return out.reshape(B, 3 * D)


# speedup vs baseline: 1.2059x; 1.0907x over previous
import functools

import jax
import jax.numpy as jnp
import numpy as np
from jax import lax
from jax.experimental import pallas as pl
from jax.experimental.pallas import tpu as pltpu
from jax.experimental.pallas import tpu_sc as plsc

B = 16384
D = 32
NUM_LOC = 1000
NUM_LVL = 10
NC = 2
NS = 16
NW = NC * NS
BPW = B // NW
CHUNK = 128
NCHUNK = BPW // CHUNK

_mesh = plsc.VectorSubcoreMesh(core_axis_name="c", subcore_axis_name="s")

_OUT_IDX = tuple(
    (4 * np.arange(B, dtype=np.int32) + k).reshape(NW * NCHUNK, CHUNK)
    for k in range(3)
)


@functools.partial(
    pl.kernel,
    mesh=_mesh,
    compiler_params=pltpu.CompilerParams(use_tc_tiling_on_sc=False),
    out_type=jax.ShapeDtypeStruct((4 * B, D), jnp.float32),
    scratch_types=[
        pltpu.VMEM((BPW,), jnp.int32),
        pltpu.VMEM((BPW,), jnp.int32),
        pltpu.VMEM((BPW,), jnp.int32),
        pltpu.VMEM((NCHUNK, CHUNK), jnp.int32),
        pltpu.VMEM((NCHUNK, CHUNK), jnp.int32),
        pltpu.VMEM((NCHUNK, CHUNK), jnp.int32),
        pltpu.VMEM((BPW, D), jnp.float32),
        pltpu.VMEM((BPW, D), jnp.float32),
        pltpu.VMEM((BPW, D), jnp.float32),
        pltpu.VMEM_SHARED((NUM_LOC + 1, D), jnp.float32),
        pltpu.VMEM_SHARED((NUM_LVL + 1, D), jnp.float32),
        pltpu.SemaphoreType.DMA,
        pltpu.SemaphoreType.DMA,
    ],
)
def _emb_kernel(title_idx_hbm, loc_idx_hbm, lvl_idx_hbm,
                oidx_t_hbm, oidx_l_hbm, oidx_v_hbm,
                title_tab, loc_tab, lvl_tab, out_hbm,
                tidx_v, lidx_v, vidx_v, toidx_v, loidx_v, voidx_v,
                trows, lrows, vrows, loc_sp, lvl_sp, sem_g, sem_s):
    wid = lax.axis_index("s") * NC + lax.axis_index("c")
    base = wid * BPW
    row0 = wid * NCHUNK

    pltpu.sync_copy(title_idx_hbm.at[pl.ds(base, BPW)], tidx_v)
    pltpu.sync_copy(loc_idx_hbm.at[pl.ds(base, BPW)], lidx_v)
    pltpu.sync_copy(lvl_idx_hbm.at[pl.ds(base, BPW)], vidx_v)
    pltpu.sync_copy(oidx_t_hbm.at[pl.ds(row0, NCHUNK)], toidx_v)
    pltpu.sync_copy(oidx_l_hbm.at[pl.ds(row0, NCHUNK)], loidx_v)
    pltpu.sync_copy(oidx_v_hbm.at[pl.ds(row0, NCHUNK)], voidx_v)

    gathers = []
    for j in range(NCHUNK):
        gathers.append(pltpu.async_copy(
            title_tab.at[tidx_v.at[pl.ds(j * CHUNK, CHUNK)]],
            trows.at[pl.ds(j * CHUNK, CHUNK)], sem_g))

    @pl.when(lax.axis_index("s") == 0)
    def _stage():
        pltpu.sync_copy(loc_tab, loc_sp)
        pltpu.sync_copy(lvl_tab, lvl_sp)
    plsc.subcore_barrier()

    for j in range(NCHUNK):
        gathers.append(pltpu.async_copy(
            loc_sp.at[lidx_v.at[pl.ds(j * CHUNK, CHUNK)]],
            lrows.at[pl.ds(j * CHUNK, CHUNK)], sem_g))
        gathers.append(pltpu.async_copy(
            lvl_sp.at[vidx_v.at[pl.ds(j * CHUNK, CHUNK)]],
            vrows.at[pl.ds(j * CHUNK, CHUNK)], sem_g))
    for c in gathers:
        c.wait()

    scatters = []
    for j in range(NCHUNK):
        scatters.append(pltpu.async_copy(
            trows.at[pl.ds(j * CHUNK, CHUNK)], out_hbm.at[toidx_v.at[j]], sem_s))
        scatters.append(pltpu.async_copy(
            lrows.at[pl.ds(j * CHUNK, CHUNK)], out_hbm.at[loidx_v.at[j]], sem_s))
        scatters.append(pltpu.async_copy(
            vrows.at[pl.ds(j * CHUNK, CHUNK)], out_hbm.at[voidx_v.at[j]], sem_s))
    for c in scatters:
        c.wait()


def kernel(movie_title, location, level, title_table, location_table, level_table):
    t_idx = movie_title.astype(jnp.int32)
    l_idx = location.astype(jnp.int32)
    v_idx = level.astype(jnp.int32)
    out = _emb_kernel(t_idx, l_idx, v_idx,
                      _OUT_IDX[0], _OUT_IDX[1], _OUT_IDX[2],
                      title_table, location_table, level_table)
    return out.reshape(B, 4 * D)[:, :3 * D]


# speedup vs baseline: 1.2383x; 1.0269x over previous
import functools

import jax
import jax.numpy as jnp
import numpy as np
from jax import lax
from jax.experimental import pallas as pl
from jax.experimental.pallas import tpu as pltpu
from jax.experimental.pallas import tpu_sc as plsc

B = 16384
D = 32
NUM_LOC = 1000
NUM_LVL = 10
NC = 2
NS = 16
NW = NC * NS
BPW = B // NW
CHUNK = 512
NCHUNK = BPW // CHUNK

_mesh = plsc.VectorSubcoreMesh(core_axis_name="c", subcore_axis_name="s")

_OUT_IDX = tuple(
    (4 * np.arange(B, dtype=np.int32) + k).reshape(NW * NCHUNK, CHUNK)
    for k in range(3)
)


@functools.partial(
    pl.kernel,
    mesh=_mesh,
    compiler_params=pltpu.CompilerParams(use_tc_tiling_on_sc=False),
    out_type=jax.ShapeDtypeStruct((4 * B, D), jnp.float32),
    scratch_types=[
        pltpu.VMEM((BPW,), jnp.int32),
        pltpu.VMEM((BPW,), jnp.int32),
        pltpu.VMEM((BPW,), jnp.int32),
        pltpu.VMEM((NCHUNK, CHUNK), jnp.int32),
        pltpu.VMEM((NCHUNK, CHUNK), jnp.int32),
        pltpu.VMEM((NCHUNK, CHUNK), jnp.int32),
        pltpu.VMEM((BPW, D), jnp.float32),
        pltpu.VMEM((BPW, D), jnp.float32),
        pltpu.VMEM((BPW, D), jnp.float32),
        pltpu.VMEM_SHARED((NUM_LOC + 1, D), jnp.float32),
        pltpu.VMEM_SHARED((NUM_LVL + 1, D), jnp.float32),
        pltpu.SemaphoreType.DMA,
        pltpu.SemaphoreType.DMA,
    ],
)
def _emb_kernel(title_idx_hbm, loc_idx_hbm, lvl_idx_hbm,
                oidx_t_hbm, oidx_l_hbm, oidx_v_hbm,
                title_tab, loc_tab, lvl_tab, out_hbm,
                tidx_v, lidx_v, vidx_v, toidx_v, loidx_v, voidx_v,
                trows, lrows, vrows, loc_sp, lvl_sp, sem_g, sem_s):
    wid = lax.axis_index("s") * NC + lax.axis_index("c")
    base = wid * BPW
    row0 = wid * NCHUNK

    pltpu.sync_copy(title_idx_hbm.at[pl.ds(base, BPW)], tidx_v)
    pltpu.sync_copy(loc_idx_hbm.at[pl.ds(base, BPW)], lidx_v)
    pltpu.sync_copy(lvl_idx_hbm.at[pl.ds(base, BPW)], vidx_v)
    pltpu.sync_copy(oidx_t_hbm.at[pl.ds(row0, NCHUNK)], toidx_v)
    pltpu.sync_copy(oidx_l_hbm.at[pl.ds(row0, NCHUNK)], loidx_v)
    pltpu.sync_copy(oidx_v_hbm.at[pl.ds(row0, NCHUNK)], voidx_v)

    gathers = []
    for j in range(NCHUNK):
        gathers.append(pltpu.async_copy(
            title_tab.at[tidx_v.at[pl.ds(j * CHUNK, CHUNK)]],
            trows.at[pl.ds(j * CHUNK, CHUNK)], sem_g))

    @pl.when(lax.axis_index("s") == 0)
    def _stage():
        pltpu.sync_copy(loc_tab, loc_sp)
        pltpu.sync_copy(lvl_tab, lvl_sp)
    plsc.subcore_barrier()

    for j in range(NCHUNK):
        gathers.append(pltpu.async_copy(
            loc_sp.at[lidx_v.at[pl.ds(j * CHUNK, CHUNK)]],
            lrows.at[pl.ds(j * CHUNK, CHUNK)], sem_g))
        gathers.append(pltpu.async_copy(
            lvl_sp.at[vidx_v.at[pl.ds(j * CHUNK, CHUNK)]],
            vrows.at[pl.ds(j * CHUNK, CHUNK)], sem_g))
    for c in gathers:
        c.wait()

    scatters = []
    for j in range(NCHUNK):
        scatters.append(pltpu.async_copy(
            trows.at[pl.ds(j * CHUNK, CHUNK)], out_hbm.at[toidx_v.at[j]], sem_s))
        scatters.append(pltpu.async_copy(
            lrows.at[pl.ds(j * CHUNK, CHUNK)], out_hbm.at[loidx_v.at[j]], sem_s))
        scatters.append(pltpu.async_copy(
            vrows.at[pl.ds(j * CHUNK, CHUNK)], out_hbm.at[voidx_v.at[j]], sem_s))
    for c in scatters:
        c.wait()


def kernel(movie_title, location, level, title_table, location_table, level_table):
    t_idx = movie_title.astype(jnp.int32)
    l_idx = location.astype(jnp.int32)
    v_idx = level.astype(jnp.int32)
    out = _emb_kernel(t_idx, l_idx, v_idx,
                      _OUT_IDX[0], _OUT_IDX[1], _OUT_IDX[2],
                      title_table, location_table, level_table)
    return out.reshape(B, 4 * D)[:, :3 * D]


# speedup vs baseline: 1.2412x; 1.0024x over previous
import functools

import jax
import jax.numpy as jnp
import numpy as np
from jax import lax
from jax.experimental import pallas as pl
from jax.experimental.pallas import tpu as pltpu
from jax.experimental.pallas import tpu_sc as plsc

B = 16384
D = 32
NUM_LOC = 1000
NUM_LVL = 10
NC = 2
NS = 16
NW = NC * NS
BPW = B // NW
CHUNK = 512
NCHUNK = BPW // CHUNK

_mesh = plsc.VectorSubcoreMesh(core_axis_name="c", subcore_axis_name="s")

_OUT_IDX = tuple(
    (4 * np.arange(B, dtype=np.int32) + k).reshape(NW * NCHUNK, CHUNK)
    for k in range(3)
)


@functools.partial(
    pl.kernel,
    mesh=_mesh,
    compiler_params=pltpu.CompilerParams(use_tc_tiling_on_sc=False),
    out_type=jax.ShapeDtypeStruct((4 * B, D), jnp.float32),
    scratch_types=[
        pltpu.VMEM((BPW,), jnp.int32),
        pltpu.VMEM((BPW,), jnp.int32),
        pltpu.VMEM((BPW,), jnp.int32),
        pltpu.VMEM((NCHUNK, CHUNK), jnp.int32),
        pltpu.VMEM((NCHUNK, CHUNK), jnp.int32),
        pltpu.VMEM((NCHUNK, CHUNK), jnp.int32),
        pltpu.VMEM((BPW, D), jnp.float32),
        pltpu.VMEM((BPW, D), jnp.float32),
        pltpu.VMEM((BPW, D), jnp.float32),
        pltpu.VMEM_SHARED((NUM_LOC + 1, D), jnp.float32),
        pltpu.VMEM_SHARED((NUM_LVL + 1, D), jnp.float32),
        pltpu.SemaphoreType.DMA,
        pltpu.SemaphoreType.DMA,
        pltpu.SemaphoreType.DMA,
        pltpu.SemaphoreType.DMA,
    ],
)
def _emb_kernel(title_idx_hbm, loc_idx_hbm, lvl_idx_hbm,
                oidx_t_hbm, oidx_l_hbm, oidx_v_hbm,
                title_tab, loc_tab, lvl_tab, out_hbm,
                tidx_v, lidx_v, vidx_v, toidx_v, loidx_v, voidx_v,
                trows, lrows, vrows, loc_sp, lvl_sp, sem_g, sem_l, sem_v, sem_s):
    wid = lax.axis_index("s") * NC + lax.axis_index("c")
    base = wid * BPW
    row0 = wid * NCHUNK

    pltpu.sync_copy(title_idx_hbm.at[pl.ds(base, BPW)], tidx_v)
    pltpu.sync_copy(loc_idx_hbm.at[pl.ds(base, BPW)], lidx_v)
    pltpu.sync_copy(lvl_idx_hbm.at[pl.ds(base, BPW)], vidx_v)
    pltpu.sync_copy(oidx_t_hbm.at[pl.ds(row0, NCHUNK)], toidx_v)
    pltpu.sync_copy(oidx_l_hbm.at[pl.ds(row0, NCHUNK)], loidx_v)
    pltpu.sync_copy(oidx_v_hbm.at[pl.ds(row0, NCHUNK)], voidx_v)

    gt = pltpu.async_copy(title_tab.at[tidx_v], trows, sem_g)

    @pl.when(lax.axis_index("s") == 0)
    def _stage():
        pltpu.sync_copy(loc_tab, loc_sp)
        pltpu.sync_copy(lvl_tab, lvl_sp)
    plsc.subcore_barrier()

    gl = pltpu.async_copy(loc_sp.at[lidx_v], lrows, sem_l)
    gv = pltpu.async_copy(lvl_sp.at[vidx_v], vrows, sem_v)

    gl.wait()
    sl = pltpu.async_copy(lrows, out_hbm.at[loidx_v.at[0]], sem_s)
    gv.wait()
    sv = pltpu.async_copy(vrows, out_hbm.at[voidx_v.at[0]], sem_s)
    gt.wait()
    st = pltpu.async_copy(trows, out_hbm.at[toidx_v.at[0]], sem_s)
    sl.wait()
    sv.wait()
    st.wait()


def kernel(movie_title, location, level, title_table, location_table, level_table):
    t_idx = movie_title.astype(jnp.int32)
    l_idx = location.astype(jnp.int32)
    v_idx = level.astype(jnp.int32)
    out = _emb_kernel(t_idx, l_idx, v_idx,
                      _OUT_IDX[0], _OUT_IDX[1], _OUT_IDX[2],
                      title_table, location_table, level_table)
    return out.reshape(B, 4 * D)[:, :3 * D]


# speedup vs baseline: 1.2843x; 1.0347x over previous
import functools

import jax
import jax.numpy as jnp
import numpy as np
from jax import lax
from jax.experimental import pallas as pl
from jax.experimental.pallas import tpu as pltpu
from jax.experimental.pallas import tpu_sc as plsc

B = 16384
D = 32
NUM_LOC = 1000
NUM_LVL = 10
NC = 2
NS = 16
NW = NC * NS
BPW = B // NW
CHUNK = 512
NCHUNK = BPW // CHUNK

_mesh = plsc.VectorSubcoreMesh(core_axis_name="c", subcore_axis_name="s")

_OUT_IDX = tuple(
    (4 * np.arange(B, dtype=np.int32) + k).reshape(NW * NCHUNK, CHUNK)
    for k in range(3)
)


@functools.partial(
    pl.kernel,
    mesh=_mesh,
    compiler_params=pltpu.CompilerParams(use_tc_tiling_on_sc=False),
    out_type=jax.ShapeDtypeStruct((4 * B, D), jnp.float32),
    scratch_types=[
        pltpu.VMEM((BPW,), jnp.int32),
        pltpu.VMEM((BPW,), jnp.int32),
        pltpu.VMEM((BPW,), jnp.int32),
        pltpu.VMEM((NCHUNK, CHUNK), jnp.int32),
        pltpu.VMEM((NCHUNK, CHUNK), jnp.int32),
        pltpu.VMEM((NCHUNK, CHUNK), jnp.int32),
        pltpu.VMEM((BPW, D), jnp.float32),
        pltpu.VMEM((BPW, D), jnp.float32),
        pltpu.VMEM((BPW, D), jnp.float32),
        pltpu.VMEM_SHARED((NUM_LOC + 1, D), jnp.float32),
        pltpu.VMEM_SHARED((NUM_LVL + 1, D), jnp.float32),
        pltpu.SemaphoreType.DMA,
        pltpu.SemaphoreType.DMA,
        pltpu.SemaphoreType.DMA,
        pltpu.SemaphoreType.DMA,
        pltpu.SemaphoreType.DMA,
        pltpu.SemaphoreType.DMA,
        pltpu.SemaphoreType.DMA,
        pltpu.SemaphoreType.DMA,
    ],
)
def _emb_kernel(title_idx_hbm, loc_idx_hbm, lvl_idx_hbm,
                oidx_t_hbm, oidx_l_hbm, oidx_v_hbm,
                title_tab, loc_tab, lvl_tab, out_hbm,
                tidx_v, lidx_v, vidx_v, toidx_v, loidx_v, voidx_v,
                trows, lrows, vrows, loc_sp, lvl_sp,
                sem_it, sem_il, sem_iv, sem_oi, sem_g, sem_l, sem_v, sem_s):
    wid = lax.axis_index("s") * NC + lax.axis_index("c")
    base = wid * BPW
    row0 = wid * NCHUNK

    it = pltpu.async_copy(title_idx_hbm.at[pl.ds(base, BPW)], tidx_v, sem_it)
    il = pltpu.async_copy(loc_idx_hbm.at[pl.ds(base, BPW)], lidx_v, sem_il)
    iv = pltpu.async_copy(lvl_idx_hbm.at[pl.ds(base, BPW)], vidx_v, sem_iv)
    ot = pltpu.async_copy(oidx_t_hbm.at[pl.ds(row0, NCHUNK)], toidx_v, sem_oi)
    ol = pltpu.async_copy(oidx_l_hbm.at[pl.ds(row0, NCHUNK)], loidx_v, sem_oi)
    ov = pltpu.async_copy(oidx_v_hbm.at[pl.ds(row0, NCHUNK)], voidx_v, sem_oi)

    @pl.when(lax.axis_index("s") == 0)
    def _stage():
        pltpu.sync_copy(loc_tab, loc_sp)
        pltpu.sync_copy(lvl_tab, lvl_sp)

    it.wait()
    gt = pltpu.async_copy(title_tab.at[tidx_v], trows, sem_g)
    plsc.subcore_barrier()

    il.wait()
    gl = pltpu.async_copy(loc_sp.at[lidx_v], lrows, sem_l)
    iv.wait()
    gv = pltpu.async_copy(lvl_sp.at[vidx_v], vrows, sem_v)
    ot.wait()
    ol.wait()
    ov.wait()

    gl.wait()
    sl = pltpu.async_copy(lrows, out_hbm.at[loidx_v.at[0]], sem_s)
    gv.wait()
    sv = pltpu.async_copy(vrows, out_hbm.at[voidx_v.at[0]], sem_s)
    gt.wait()
    st = pltpu.async_copy(trows, out_hbm.at[toidx_v.at[0]], sem_s)
    sl.wait()
    sv.wait()
    st.wait()


def kernel(movie_title, location, level, title_table, location_table, level_table):
    t_idx = movie_title.astype(jnp.int32)
    l_idx = location.astype(jnp.int32)
    v_idx = level.astype(jnp.int32)
    out = _emb_kernel(t_idx, l_idx, v_idx,
                      _OUT_IDX[0], _OUT_IDX[1], _OUT_IDX[2],
                      title_table, location_table, level_table)
    return out.reshape(B, 4 * D)[:, :3 * D]
